# Initial kernel scaffold; baseline (speedup 1.0000x reference)
import jax
import jax.numpy as jnp
from jax.experimental import pallas as pl


def kernel(h, edge_index, snorm_n, W_self, W_func, W_att):
    raise NotImplementedError("write your pallas kernel here")


# speedup vs baseline: 6.1665x; 6.1665x over previous
import functools

import jax
import jax.numpy as jnp
from jax import lax
from jax.experimental import pallas as pl
from jax.experimental.pallas import tpu as pltpu
from jax.experimental.pallas import tpu_sc as plsc

N = 10000
E = 320000
D = 128

NPAD = 10240
CH = 128
EPAD = 323584
NSC = 2
NTILE = 16
E_PER_TILE1 = EPAD // NTILE
E_PER_TILE2 = EPAD // (NSC * NTILE)
ROWS_PER_TILE = NPAD // NTILE

BLK = 2560


def _pre_body(h_ref, ws_ref, wf_ref, wp_ref, hs_ref, z_ref, a_ref):
    hblk = h_ref[...]
    dn = (((1,), (1,)), ((), ()))
    z = lax.dot_general(hblk, wf_ref[...], dn, preferred_element_type=jnp.float32)
    hs_ref[...] = lax.dot_general(hblk, ws_ref[...], dn,
                                  preferred_element_type=jnp.float32)
    z_ref[...] = z
    a_ref[...] = lax.dot_general(wp_ref[...], z, (((0,), (1,)), ((), ())),
                                 preferred_element_type=jnp.float32)


@jax.jit
def _tc_pre(h_p, W_self, W_func, wp8):
    return pl.pallas_call(
        _pre_body,
        grid=(NPAD // BLK,),
        in_specs=[
            pl.BlockSpec((BLK, D), lambda i: (i, 0)),
            pl.BlockSpec((D, D), lambda i: (0, 0)),
            pl.BlockSpec((D, D), lambda i: (0, 0)),
            pl.BlockSpec((D, 8), lambda i: (0, 0)),
        ],
        out_specs=[
            pl.BlockSpec((BLK, D), lambda i: (i, 0)),
            pl.BlockSpec((BLK, D), lambda i: (i, 0)),
            pl.BlockSpec((8, BLK), lambda i: (0, i)),
        ],
        out_shape=[
            jax.ShapeDtypeStruct((NPAD, D), jnp.float32),
            jax.ShapeDtypeStruct((NPAD, D), jnp.float32),
            jax.ShapeDtypeStruct((8, NPAD), jnp.float32),
        ],
    )(h_p, W_self, W_func, wp8)


def _sc_body(src_hbm, dst_hbm, a1_hbm, a2_hbm, z_hbm, agg_hbm,
             idx_src, idx_dst, v1, v2, vals, rows, zrows, zvec,
             denom_sp, agg_sp, sem):
    c = lax.axis_index("c")
    s = lax.axis_index("s")

    z16 = jnp.zeros((16,), jnp.float32)
    def zb_body(i, _):
        for q in range(D // 16):
            zrows[i, pl.ds(q * 16, 16)] = z16
        return 0
    lax.fori_loop(0, 32, zb_body, 0)
    def zv_body(i, _):
        zvec[pl.ds(i * 16, 16)] = z16
        return 0
    lax.fori_loop(0, ROWS_PER_TILE // 16, zv_body, 0)
    r0 = s * ROWS_PER_TILE
    def zcp_body(i, _):
        pltpu.sync_copy(zrows, agg_sp.at[pl.ds(r0 + i * 32, 32)])
        return 0
    lax.fori_loop(0, ROWS_PER_TILE // 32, zcp_body, 0)
    pltpu.sync_copy(zvec, denom_sp.at[pl.ds(r0, ROWS_PER_TILE)])
    plsc.subcore_barrier()

    g0 = s * E_PER_TILE1
    def ph1_body(k, _):
        base = g0 + k * CH
        pltpu.sync_copy(src_hbm.at[pl.ds(base, CH)], idx_src)
        pltpu.sync_copy(dst_hbm.at[pl.ds(base, CH)], idx_dst)
        cp1 = pltpu.async_copy(a1_hbm.at[idx_src], v1, sem)
        cp1.wait()
        cp2 = pltpu.async_copy(a2_hbm.at[idx_dst], v2, sem)
        cp2.wait()
        for q in range(CH // 16):
            e = v1[pl.ds(q * 16, 16)] + v2[pl.ds(q * 16, 16)]
            e = jnp.where(e >= 0.0, e, 0.01 * e)
            vals[pl.ds(q * 16, 16)] = jnp.exp(e)
        pltpu.sync_copy(vals, denom_sp.at[idx_dst], add=True)
        return 0
    lax.fori_loop(0, E_PER_TILE1 // CH, ph1_body, 0)
    plsc.subcore_barrier()

    g2 = (c * NTILE + s) * E_PER_TILE2
    def ph2(k, _):
        base = g2 + k * CH
        pltpu.sync_copy(src_hbm.at[pl.ds(base, CH)], idx_src)
        pltpu.sync_copy(dst_hbm.at[pl.ds(base, CH)], idx_dst)
        cp1 = pltpu.async_copy(a1_hbm.at[idx_src], v1, sem)
        cp1.wait()
        cp2 = pltpu.async_copy(a2_hbm.at[idx_dst], v2, sem)
        cp2.wait()
        for q in range(CH // 16):
            e = v1[pl.ds(q * 16, 16)] + v2[pl.ds(q * 16, 16)]
            e = jnp.where(e >= 0.0, e, 0.01 * e)
            vals[pl.ds(q * 16, 16)] = jnp.exp(e)
        cpd = pltpu.async_copy(denom_sp.at[idx_dst], v1, sem)
        cpd.wait()
        for q in range(CH // 16):
            vals[pl.ds(q * 16, 16)] = (vals[pl.ds(q * 16, 16)]
                                       / v1[pl.ds(q * 16, 16)])
        cpz = pltpu.async_copy(z_hbm.at[idx_src], rows, sem)
        cpz.wait()
        def scale_body(g, _):
            avec = vals[pl.ds(g * 16, 16)]
            for jj in range(16):
                ab = avec[jnp.full((16,), jj, jnp.int32)]
                j = g * 16 + jj
                for q in range(D // 16):
                    rows[j, pl.ds(q * 16, 16)] = rows[j, pl.ds(q * 16, 16)] * ab
            return 0
        lax.fori_loop(0, CH // 16, scale_body, 0)
        pltpu.sync_copy(rows, agg_sp.at[idx_dst], add=True)
        return 0
    lax.fori_loop(0, E_PER_TILE2 // CH, ph2, 0)
    plsc.subcore_barrier()

    pltpu.sync_copy(agg_sp.at[pl.ds(r0, ROWS_PER_TILE)],
                    agg_hbm.at[c, pl.ds(r0, ROWS_PER_TILE)])


@jax.jit
def _sc_gat(src_p, dst_p, a1, a2, z):
    return pl.kernel(
        _sc_body,
        out_type=jax.ShapeDtypeStruct((NSC, NPAD, D), jnp.float32),
        mesh=plsc.VectorSubcoreMesh(core_axis_name="c", subcore_axis_name="s"),
        scratch_types=[
            pltpu.VMEM((CH,), jnp.int32),
            pltpu.VMEM((CH,), jnp.int32),
            pltpu.VMEM((CH,), jnp.float32),
            pltpu.VMEM((CH,), jnp.float32),
            pltpu.VMEM((CH,), jnp.float32),
            pltpu.VMEM((CH, D), jnp.float32),
            pltpu.VMEM((32, D), jnp.float32),
            pltpu.VMEM((ROWS_PER_TILE,), jnp.float32),
            pltpu.VMEM_SHARED((NPAD,), jnp.float32),
            pltpu.VMEM_SHARED((NPAD, D), jnp.float32),
            pltpu.SemaphoreType.DMA,
        ],
    )(src_p, dst_p, a1, a2, z)


def _fin_body(h_ref, hs_ref, p_ref, out_ref):
    acc = hs_ref[...] + p_ref[0] + p_ref[1]
    out_ref[...] = h_ref[...] + jnp.maximum(acc, 0.0)


@jax.jit
def _tc_fin(h_p, hs, agg):
    return pl.pallas_call(
        _fin_body,
        grid=(NPAD // BLK,),
        in_specs=[
            pl.BlockSpec((BLK, D), lambda i: (i, 0)),
            pl.BlockSpec((BLK, D), lambda i: (i, 0)),
            pl.BlockSpec((NSC, BLK, D), lambda i: (0, i, 0)),
        ],
        out_specs=pl.BlockSpec((BLK, D), lambda i: (i, 0)),
        out_shape=jax.ShapeDtypeStruct((NPAD, D), jnp.float32),
    )(h_p, hs, agg)


def kernel(h, edge_index, snorm_n, W_self, W_func, W_att):
    src = edge_index[0].astype(jnp.int32)
    dst = edge_index[1].astype(jnp.int32)
    pad_e = EPAD - E
    fill = jnp.full((pad_e,), NPAD - 1, jnp.int32)
    src_p = jnp.concatenate([src, fill])
    dst_p = jnp.concatenate([dst, fill])
    h_p = jnp.pad(h, ((0, NPAD - N), (0, 0)))
    wp8 = jnp.pad(W_att.reshape(2, D).T, ((0, 0), (0, 6)))

    hs, z, a12 = _tc_pre(h_p, W_self, W_func, wp8)
    a1 = a12[0]
    a2 = a12[1]
    agg = _sc_gat(src_p, dst_p, a1, a2, z)
    out = _tc_fin(h_p, hs, agg)
    return out[:N]


# speedup vs baseline: 10.7174x; 1.7380x over previous
import functools

import jax
import jax.numpy as jnp
from jax import lax
from jax.experimental import pallas as pl
from jax.experimental.pallas import tpu as pltpu
from jax.experimental.pallas import tpu_sc as plsc

N = 10000
E = 320000
D = 128

NPAD = 10240
CH = 128
EPAD = 323584
NSC = 2
NTILE = 16
NCHUNK = EPAD // (NSC * NTILE * CH)
ROWS_PER_TILE = NPAD // NTILE

BLK = 2560


def _pre_body(h_ref, ws_ref, wf_ref, wp_ref, hs_ref, z_ref, a_ref):
    hblk = h_ref[...]
    dn = (((1,), (1,)), ((), ()))
    z = lax.dot_general(hblk, wf_ref[...], dn, preferred_element_type=jnp.float32)
    hs_ref[...] = lax.dot_general(hblk, ws_ref[...], dn,
                                  preferred_element_type=jnp.float32)
    z_ref[...] = z
    a_ref[...] = lax.dot_general(wp_ref[...], z, (((0,), (1,)), ((), ())),
                                 preferred_element_type=jnp.float32)


@jax.jit
def _tc_pre(h_p, W_self, W_func, wp8):
    return pl.pallas_call(
        _pre_body,
        grid=(NPAD // BLK,),
        in_specs=[
            pl.BlockSpec((BLK, D), lambda i: (i, 0)),
            pl.BlockSpec((D, D), lambda i: (0, 0)),
            pl.BlockSpec((D, D), lambda i: (0, 0)),
            pl.BlockSpec((D, 8), lambda i: (0, 0)),
        ],
        out_specs=[
            pl.BlockSpec((BLK, D), lambda i: (i, 0)),
            pl.BlockSpec((BLK, D), lambda i: (i, 0)),
            pl.BlockSpec((8, BLK), lambda i: (0, i)),
        ],
        out_shape=[
            jax.ShapeDtypeStruct((NPAD, D), jnp.float32),
            jax.ShapeDtypeStruct((NPAD, D), jnp.float32),
            jax.ShapeDtypeStruct((8, NPAD), jnp.float32),
        ],
    )(h_p, W_self, W_func, wp8)


def _sc_body(sd_hbm, a1_hbm, a2_hbm, z_hbm, agg_hbm, den_hbm,
             idx2, v1, v2, vals, rows, zrows, zvec,
             denom_sp, agg_sp, sem):
    c = lax.axis_index("c")
    s = lax.axis_index("s")

    z16 = jnp.zeros((16,), jnp.float32)
    def zb_body(i, _):
        for q in range(D // 16):
            zrows[i, pl.ds(q * 16, 16)] = z16
        return 0
    lax.fori_loop(0, 32, zb_body, 0)
    def zv_body(i, _):
        zvec[pl.ds(i * 16, 16)] = z16
        return 0
    lax.fori_loop(0, ROWS_PER_TILE // 16, zv_body, 0)
    r0 = s * ROWS_PER_TILE
    def zcp_body(i, _):
        pltpu.sync_copy(zrows, agg_sp.at[pl.ds(r0 + i * 32, 32)])
        return 0
    lax.fori_loop(0, ROWS_PER_TILE // 32, zcp_body, 0)
    pltpu.sync_copy(zvec, denom_sp.at[pl.ds(r0, ROWS_PER_TILE)])
    plsc.subcore_barrier()

    k0 = (c * NTILE + s) * NCHUNK
    def ch_body(k, _):
        pltpu.sync_copy(sd_hbm.at[k0 + k], idx2)
        cp1 = pltpu.async_copy(a1_hbm.at[idx2.at[0]], v1, sem)
        cp1.wait()
        cp2 = pltpu.async_copy(a2_hbm.at[idx2.at[1]], v2, sem)
        cp2.wait()
        for q in range(CH // 16):
            e = v1[pl.ds(q * 16, 16)] + v2[pl.ds(q * 16, 16)]
            e = jnp.where(e >= 0.0, e, 0.01 * e)
            vals[pl.ds(q * 16, 16)] = jnp.exp(e)
        pltpu.sync_copy(vals, denom_sp.at[idx2.at[1]], add=True)
        cpz = pltpu.async_copy(z_hbm.at[idx2.at[0]], rows, sem)
        cpz.wait()
        def scale_body(g, _):
            avec = vals[pl.ds(g * 16, 16)]
            for jj in range(16):
                ab = avec[jnp.full((16,), jj, jnp.int32)]
                j = g * 16 + jj
                for q in range(D // 16):
                    rows[j, pl.ds(q * 16, 16)] = rows[j, pl.ds(q * 16, 16)] * ab
            return 0
        lax.fori_loop(0, CH // 16, scale_body, 0)
        pltpu.sync_copy(rows, agg_sp.at[idx2.at[1]], add=True)
        return 0
    lax.fori_loop(0, NCHUNK, ch_body, 0)
    plsc.subcore_barrier()

    pltpu.sync_copy(agg_sp.at[pl.ds(r0, ROWS_PER_TILE)],
                    agg_hbm.at[c, pl.ds(r0, ROWS_PER_TILE)])
    pltpu.sync_copy(denom_sp.at[pl.ds(r0, ROWS_PER_TILE)],
                    den_hbm.at[c, pl.ds(r0, ROWS_PER_TILE)])


@jax.jit
def _sc_gat(sd, a1, a2, z):
    return pl.kernel(
        _sc_body,
        out_type=[
            jax.ShapeDtypeStruct((NSC, NPAD, D), jnp.float32),
            jax.ShapeDtypeStruct((NSC, NPAD), jnp.float32),
        ],
        mesh=plsc.VectorSubcoreMesh(core_axis_name="c", subcore_axis_name="s"),
        scratch_types=[
            pltpu.VMEM((2, CH), jnp.int32),
            pltpu.VMEM((CH,), jnp.float32),
            pltpu.VMEM((CH,), jnp.float32),
            pltpu.VMEM((CH,), jnp.float32),
            pltpu.VMEM((CH, D), jnp.float32),
            pltpu.VMEM((32, D), jnp.float32),
            pltpu.VMEM((ROWS_PER_TILE,), jnp.float32),
            pltpu.VMEM_SHARED((NPAD,), jnp.float32),
            pltpu.VMEM_SHARED((NPAD, D), jnp.float32),
            pltpu.SemaphoreType.DMA,
        ],
    )(sd, a1, a2, z)


def _fin_body(h_ref, hs_ref, p_ref, d0_ref, d1_ref, out_ref):
    dt = d0_ref[...] + d1_ref[...]
    dinv = jnp.where(dt > 0.0, 1.0 / dt, 0.0)
    acc = hs_ref[...] + (p_ref[0] + p_ref[1]) * dinv
    out_ref[...] = h_ref[...] + jnp.maximum(acc, 0.0)


@jax.jit
def _tc_fin(h_p, hs, agg, d0, d1):
    return pl.pallas_call(
        _fin_body,
        grid=(NPAD // BLK,),
        in_specs=[
            pl.BlockSpec((BLK, D), lambda i: (i, 0)),
            pl.BlockSpec((BLK, D), lambda i: (i, 0)),
            pl.BlockSpec((NSC, BLK, D), lambda i: (0, i, 0)),
            pl.BlockSpec((BLK, 1), lambda i: (i, 0)),
            pl.BlockSpec((BLK, 1), lambda i: (i, 0)),
        ],
        out_specs=pl.BlockSpec((BLK, D), lambda i: (i, 0)),
        out_shape=jax.ShapeDtypeStruct((NPAD, D), jnp.float32),
    )(h_p, hs, agg, d0, d1)


def kernel(h, edge_index, snorm_n, W_self, W_func, W_att):
    src = edge_index[0].astype(jnp.int32)
    dst = edge_index[1].astype(jnp.int32)
    pad_e = EPAD - E
    fill = jnp.full((pad_e,), NPAD - 1, jnp.int32)
    src_p = jnp.concatenate([src, fill])
    dst_p = jnp.concatenate([dst, fill])
    sd = jnp.stack([src_p.reshape(-1, CH), dst_p.reshape(-1, CH)], axis=1)
    h_p = jnp.pad(h, ((0, NPAD - N), (0, 0)))
    wp8 = jnp.pad(W_att.reshape(2, D).T, ((0, 0), (0, 6)))

    hs, z, a12 = _tc_pre(h_p, W_self, W_func, wp8)
    a1 = a12[0]
    a2 = a12[1]
    agg, den = _sc_gat(sd, a1, a2, z)
    out = _tc_fin(h_p, hs, agg, den[0][:, None], den[1][:, None])
    return out[:N]
